# baseline (speedup 1.0000x reference)
import functools

import jax
import jax.numpy as jnp
from jax import lax
from jax.experimental import pallas as pl
from jax.experimental.pallas import tpu as pltpu
from jax.experimental.pallas import tpu_sc as plsc


N_NODES = 10000
N_EDGES = 320000
D_NODE = 128
D_EDGE = 16
D_HID = 128
D_HALF = D_HID // 2

NC = 2
NS = 16
EC = N_EDGES
EPT = EC // NS
B = 80
NSTEPS = EPT // B

ROWS_PER_TILE = 624
TAIL_BASE = NS * ROWS_PER_TILE
TAIL_ROWS = N_NODES - TAIL_BASE


def _leaky(x):
    return jnp.where(x >= 0, x, 0.01 * x)


def _msg_body(node_ref, w_ref, out_ref):
    out_ref[...] = jnp.dot(node_ref[...], w_ref[0],
                           preferred_element_type=jnp.float32)


def _msg_call(node, w_node):
    blk = 1000
    nblk = N_NODES // blk
    w_split = jnp.stack([w_node[:, :D_HALF], w_node[:, D_HALF:]])
    return pl.pallas_call(
        _msg_body,
        grid=(nblk, NC),
        in_specs=[
            pl.BlockSpec((blk, D_NODE), lambda i, c: (i, 0)),
            pl.BlockSpec((1, D_NODE, D_HALF), lambda i, c: (c, 0, 0)),
        ],
        out_specs=pl.BlockSpec((blk, D_HALF), lambda i, c: (c * nblk + i, 0)),
        out_shape=jax.ShapeDtypeStruct((NC * N_NODES, D_HALF), jnp.float32),
    )(node, w_split)


MLP_BLK = 32000
PAIR_OFF = MLP_BLK // 2


def _mlp_body(edget_ref, w1_ref, b1_ref, w2_ref, b2_ref, out_ref):
    h = lax.dot_general(edget_ref[...], w1_ref[...],
                        dimension_numbers=(((0,), (0,)), ((), ())),
                        preferred_element_type=jnp.float32) + b1_ref[...]
    h = _leaky(h)
    h = jnp.dot(h, w2_ref[...], preferred_element_type=jnp.float32) + b2_ref[...]
    h = _leaky(h)
    lob = lax.bitcast_convert_type(h[:PAIR_OFF], jnp.uint32)
    hib = lax.bitcast_convert_type(h[PAIR_OFF:], jnp.uint32)
    half = jnp.uint32(0x8000)
    w = ((lob + half) >> 16) | (((hib + half) >> 16) << 16)
    out_ref[...] = lax.bitcast_convert_type(w, jnp.int32)


def _mlp_call(edge_t, w1, b1, w2, b2):
    blk = MLP_BLK
    return pl.pallas_call(
        _mlp_body,
        grid=(EC // blk,),
        in_specs=[
            pl.BlockSpec((D_EDGE, blk), lambda i: (0, i)),
            pl.BlockSpec((D_EDGE, D_HID), lambda i: (0, 0)),
            pl.BlockSpec((1, D_HID), lambda i: (0, 0)),
            pl.BlockSpec((D_HID, D_HID), lambda i: (0, 0)),
            pl.BlockSpec((1, D_HID), lambda i: (0, 0)),
        ],
        out_specs=pl.BlockSpec((blk // 2, D_HID), lambda i: (i, 0)),
        out_shape=jax.ShapeDtypeStruct((EC // 2, D_HID), jnp.int32),
    )(edge_t, w1, b1.reshape(1, D_HID), w2, b2.reshape(1, D_HID))


def _sc_body(msg_hbm, e_hbm, idx_hbm, seg_hbm, zeros_hbm, out_hbm,
             idx2d, seg2d, rows0, rows1, ev0, ev1, prod0, prod1,
             acc, sem_g0, sem_g1, sem_e0, sem_e1):
    cid = lax.axis_index("c")
    sid = lax.axis_index("s")

    zbase = sid * ROWS_PER_TILE
    pltpu.sync_copy(zeros_hbm.at[pl.ds(zbase, ROWS_PER_TILE)],
                    acc.at[pl.ds(zbase, ROWS_PER_TILE)])

    @pl.when(sid == NS - 1)
    def _zero_tail():
        pltpu.sync_copy(zeros_hbm.at[pl.ds(TAIL_BASE, TAIL_ROWS)],
                        acc.at[pl.ds(TAIL_BASE, TAIL_ROWS)])

    pltpu.sync_copy(idx_hbm.at[sid], idx2d)
    pltpu.sync_copy(seg_hbm.at[sid], seg2d)
    roff = cid * N_NODES

    def bias_row(r, c):
        for j in range(B // 16):
            sl = pl.ds(j * 16, 16)
            idx2d[r, sl] = idx2d[r, sl] + roff
        return c
    lax.fori_loop(0, NSTEPS, bias_row, 0)
    plsc.subcore_barrier()

    epbase = sid * (EPT // 2)
    coff = cid * D_HALF

    def start_gather(t, rowbuf, sem):
        pltpu.async_copy(msg_hbm.at[idx2d.at[t]], rowbuf, sem)

    def wait_gather(t, rowbuf, sem):
        pltpu.make_async_copy(msg_hbm.at[idx2d.at[t]], rowbuf, sem).wait()

    def start_e(t, ebuf, sem):
        pltpu.async_copy(
            e_hbm.at[pl.ds(epbase + t * (B // 2), B // 2),
                     pl.ds(coff, D_HALF)], ebuf, sem)

    def wait_e(t, ebuf, sem):
        pltpu.make_async_copy(
            e_hbm.at[pl.ds(epbase + t * (B // 2), B // 2),
                     pl.ds(coff, D_HALF)], ebuf, sem).wait()

    def mul(rowbuf, ebuf, prodbuf):
        def mul_pair(p, c):
            for j in range(D_HALF // 16):
                sl = pl.ds(j * 16, 16)
                w = ebuf[p, sl]
                x = lax.bitcast_convert_type(w << 16, jnp.float32)
                y = lax.bitcast_convert_type(w & jnp.int32(-65536), jnp.float32)
                prodbuf[2 * p, sl] = rowbuf[2 * p, sl] * x
                prodbuf[2 * p + 1, sl] = rowbuf[2 * p + 1, sl] * y
            return c
        lax.fori_loop(0, B // 2, mul_pair, 0)

    def consume(t, rowbuf, ebuf, prodbuf, sem_g, sem_e, refill):
        wait_gather(t, rowbuf, sem_g)
        wait_e(t, ebuf, sem_e)
        mul(rowbuf, ebuf, prodbuf)

        @pl.when(refill)
        def _refill():
            start_gather(t + 2, rowbuf, sem_g)
            start_e(t + 2, ebuf, sem_e)

        pltpu.sync_copy(prodbuf, acc.at[seg2d.at[t]], add=True)

    start_gather(0, rows0, sem_g0)
    start_e(0, ev0, sem_e0)
    start_gather(1, rows1, sem_g1)
    start_e(1, ev1, sem_e1)

    PAIRS = NSTEPS // 2

    def pair(k, carry):
        t0 = 2 * k
        refill = k < PAIRS - 1
        consume(t0, rows0, ev0, prod0, sem_g0, sem_e0, refill)
        consume(t0 + 1, rows1, ev1, prod1, sem_g1, sem_e1, refill)
        return carry

    lax.fori_loop(0, PAIRS, pair, 0)
    plsc.subcore_barrier()

    pltpu.sync_copy(acc.at[pl.ds(zbase, ROWS_PER_TILE)],
                    out_hbm.at[pl.ds(zbase, ROWS_PER_TILE), pl.ds(coff, D_HALF)])

    @pl.when(sid == NS - 1)
    def _dump_tail():
        pltpu.sync_copy(
            acc.at[pl.ds(TAIL_BASE, TAIL_ROWS)],
            out_hbm.at[pl.ds(TAIL_BASE, TAIL_ROWS), pl.ds(coff, D_HALF)])


def _sc_call(msg2, e, idx3, seg3, zeros):
    mesh = plsc.VectorSubcoreMesh(core_axis_name="c", subcore_axis_name="s")
    kfn = functools.partial(
        pl.kernel,
        mesh=mesh,
        compiler_params=pltpu.CompilerParams(use_tc_tiling_on_sc=False),
        out_type=jax.ShapeDtypeStruct((N_NODES, D_HID), jnp.float32),
        scratch_types=[
            pltpu.VMEM((NSTEPS, B), jnp.int32),
            pltpu.VMEM((NSTEPS, B), jnp.int32),
            pltpu.VMEM((B, D_HALF), jnp.float32),
            pltpu.VMEM((B, D_HALF), jnp.float32),
            pltpu.VMEM((B // 2, D_HALF), jnp.int32),
            pltpu.VMEM((B // 2, D_HALF), jnp.int32),
            pltpu.VMEM((B, D_HALF), jnp.float32),
            pltpu.VMEM((B, D_HALF), jnp.float32),
            pltpu.VMEM_SHARED((N_NODES, D_HALF), jnp.float32),
            pltpu.SemaphoreType.DMA,
            pltpu.SemaphoreType.DMA,
            pltpu.SemaphoreType.DMA,
            pltpu.SemaphoreType.DMA,
        ],
    )(_sc_body)
    return kfn(msg2, e, idx3, seg3, zeros)


def _cat_body(a_ref, b_ref, out_ref):
    out_ref[...] = jnp.concatenate([a_ref[...], b_ref[...]], axis=1)


def _cat_call(partial):
    blk = 1000
    nblk = N_NODES // blk
    return pl.pallas_call(
        _cat_body,
        grid=(nblk,),
        in_specs=[
            pl.BlockSpec((blk, D_HALF), lambda i: (i, 0)),
            pl.BlockSpec((blk, D_HALF), lambda i: (i + N_NODES // blk, 0)),
        ],
        out_specs=pl.BlockSpec((blk, D_HID), lambda i: (i, 0)),
        out_shape=jax.ShapeDtypeStruct((N_NODES, D_HID), jnp.float32),
    )(partial, partial)


def kernel(node, edge, seg_i, idx_j, W_node, W_e1, b_e1, W_e2, b_e2):
    idx32 = idx_j.astype(jnp.int32)
    seg32 = seg_i.astype(jnp.int32)
    q = jnp.arange(N_EDGES // 2, dtype=jnp.int32)
    e_lo = (q // PAIR_OFF) * MLP_BLK + (q % PAIR_OFF)
    perm = jnp.stack([e_lo, e_lo + PAIR_OFF], axis=1).reshape(-1)
    idxp = jnp.take(idx32, perm)
    segp = jnp.take(seg32, perm)
    msg2 = _msg_call(node, W_node)
    e = _mlp_call(edge.T, W_e1, b_e1, W_e2, b_e2)
    zeros = jnp.zeros((N_NODES, D_HALF), jnp.float32)
    return _sc_call(msg2, e, idxp.reshape(NS, NSTEPS, B),
                    segp.reshape(NS, NSTEPS, B), zeros)


# speedup vs baseline: 1.9219x; 1.9219x over previous
import functools

import jax
import jax.numpy as jnp
from jax import lax
from jax.experimental import pallas as pl
from jax.experimental.pallas import tpu as pltpu
from jax.experimental.pallas import tpu_sc as plsc


N_NODES = 10000
N_EDGES = 320000
D_NODE = 128
D_EDGE = 16
D_HID = 128
D_HALF = D_HID // 2

NC = 2
NS = 16
EC = N_EDGES
EPT = EC // NS
B = 80
NSTEPS = EPT // B

ROWS_PER_TILE = 624
TAIL_BASE = NS * ROWS_PER_TILE
TAIL_ROWS = N_NODES - TAIL_BASE


def _leaky(x):
    return jnp.where(x >= 0, x, 0.01 * x)


def _msg_body(node_ref, w_ref, out_ref):
    out_ref[...] = jnp.dot(node_ref[...], w_ref[0],
                           preferred_element_type=jnp.float32)


def _msg_call(node, w_node):
    blk = 1000
    nblk = N_NODES // blk
    w_split = jnp.stack([w_node[:, :D_HALF], w_node[:, D_HALF:]])
    return pl.pallas_call(
        _msg_body,
        grid=(nblk, NC),
        in_specs=[
            pl.BlockSpec((blk, D_NODE), lambda i, c: (i, 0)),
            pl.BlockSpec((1, D_NODE, D_HALF), lambda i, c: (c, 0, 0)),
        ],
        out_specs=pl.BlockSpec((blk, D_HALF), lambda i, c: (c * nblk + i, 0)),
        out_shape=jax.ShapeDtypeStruct((NC * N_NODES, D_HALF), jnp.float32),
    )(node, w_split)


def _mlp_body(edget_ref, w1_ref, b1_ref, w2_ref, b2_ref, out_ref):
    h = lax.dot_general(edget_ref[...], w1_ref[...],
                        dimension_numbers=(((0,), (0,)), ((), ())),
                        preferred_element_type=jnp.float32) + b1_ref[...]
    h = _leaky(h)
    h = jnp.dot(h, w2_ref[...], preferred_element_type=jnp.float32) + b2_ref[...]
    out_ref[...] = _leaky(h)


def _mlp_call(edge_t, w1, b1, w2, b2):
    blk = 32000
    return pl.pallas_call(
        _mlp_body,
        grid=(EC // blk,),
        in_specs=[
            pl.BlockSpec((D_EDGE, blk), lambda i: (0, i)),
            pl.BlockSpec((D_EDGE, D_HID), lambda i: (0, 0)),
            pl.BlockSpec((1, D_HID), lambda i: (0, 0)),
            pl.BlockSpec((D_HID, D_HID), lambda i: (0, 0)),
            pl.BlockSpec((1, D_HID), lambda i: (0, 0)),
        ],
        out_specs=pl.BlockSpec((blk, D_HID), lambda i: (i, 0)),
        out_shape=jax.ShapeDtypeStruct((EC, D_HID), jnp.float32),
    )(edge_t, w1, b1.reshape(1, D_HID), w2, b2.reshape(1, D_HID))


def _sc_body(msg_hbm, e_hbm, idx_hbm, seg_hbm, zeros_hbm, out_hbm,
             idx2d, seg2d, rows0, rows1, ev0, ev1, prod0, prod1,
             acc, sem_g0, sem_g1, sem_e0, sem_e1):
    cid = lax.axis_index("c")
    sid = lax.axis_index("s")

    zbase = sid * ROWS_PER_TILE
    pltpu.sync_copy(zeros_hbm.at[pl.ds(zbase, ROWS_PER_TILE)],
                    acc.at[pl.ds(zbase, ROWS_PER_TILE)])

    @pl.when(sid == NS - 1)
    def _zero_tail():
        pltpu.sync_copy(zeros_hbm.at[pl.ds(TAIL_BASE, TAIL_ROWS)],
                        acc.at[pl.ds(TAIL_BASE, TAIL_ROWS)])

    pltpu.sync_copy(idx_hbm.at[sid], idx2d)
    pltpu.sync_copy(seg_hbm.at[sid], seg2d)
    roff = cid * N_NODES

    def bias_row(r, c):
        for j in range(B // 16):
            sl = pl.ds(j * 16, 16)
            idx2d[r, sl] = idx2d[r, sl] + roff
        return c
    lax.fori_loop(0, NSTEPS, bias_row, 0)
    plsc.subcore_barrier()

    ebase = sid * EPT
    coff = cid * D_HALF

    def start_gather(t, rowbuf, sem):
        pltpu.async_copy(msg_hbm.at[idx2d.at[t]], rowbuf, sem)

    def wait_gather(t, rowbuf, sem):
        pltpu.make_async_copy(msg_hbm.at[idx2d.at[t]], rowbuf, sem).wait()

    def start_e(t, ebuf, sem):
        pltpu.async_copy(
            e_hbm.at[pl.ds(ebase + t * B, B), pl.ds(coff, D_HALF)], ebuf, sem)

    def wait_e(t, ebuf, sem):
        pltpu.make_async_copy(
            e_hbm.at[pl.ds(ebase + t * B, B), pl.ds(coff, D_HALF)], ebuf,
            sem).wait()

    def mul(rowbuf, ebuf, prodbuf):
        def mul_row(r, c):
            for j in range(D_HALF // 16):
                sl = pl.ds(j * 16, 16)
                prodbuf[r, sl] = rowbuf[r, sl] * ebuf[r, sl]
            return c
        lax.fori_loop(0, B, mul_row, 0)

    def consume(t, rowbuf, ebuf, prodbuf, sem_g, sem_e, refill):
        wait_gather(t, rowbuf, sem_g)
        wait_e(t, ebuf, sem_e)
        mul(rowbuf, ebuf, prodbuf)

        @pl.when(refill)
        def _refill():
            start_gather(t + 2, rowbuf, sem_g)
            start_e(t + 2, ebuf, sem_e)

        pltpu.sync_copy(prodbuf, acc.at[seg2d.at[t]], add=True)

    start_gather(0, rows0, sem_g0)
    start_e(0, ev0, sem_e0)
    start_gather(1, rows1, sem_g1)
    start_e(1, ev1, sem_e1)

    PAIRS = NSTEPS // 2

    def pair(k, carry):
        t0 = 2 * k
        refill = k < PAIRS - 1
        consume(t0, rows0, ev0, prod0, sem_g0, sem_e0, refill)
        consume(t0 + 1, rows1, ev1, prod1, sem_g1, sem_e1, refill)
        return carry

    lax.fori_loop(0, PAIRS, pair, 0)
    plsc.subcore_barrier()

    pltpu.sync_copy(acc.at[pl.ds(zbase, ROWS_PER_TILE)],
                    out_hbm.at[pl.ds(zbase, ROWS_PER_TILE), pl.ds(coff, D_HALF)])

    @pl.when(sid == NS - 1)
    def _dump_tail():
        pltpu.sync_copy(
            acc.at[pl.ds(TAIL_BASE, TAIL_ROWS)],
            out_hbm.at[pl.ds(TAIL_BASE, TAIL_ROWS), pl.ds(coff, D_HALF)])


def _sc_call(msg2, e, idx3, seg3, zeros):
    mesh = plsc.VectorSubcoreMesh(core_axis_name="c", subcore_axis_name="s")
    kfn = functools.partial(
        pl.kernel,
        mesh=mesh,
        compiler_params=pltpu.CompilerParams(use_tc_tiling_on_sc=False),
        out_type=jax.ShapeDtypeStruct((N_NODES, D_HID), jnp.float32),
        scratch_types=[
            pltpu.VMEM((NSTEPS, B), jnp.int32),
            pltpu.VMEM((NSTEPS, B), jnp.int32),
            pltpu.VMEM((B, D_HALF), jnp.float32),
            pltpu.VMEM((B, D_HALF), jnp.float32),
            pltpu.VMEM((B, D_HALF), jnp.float32),
            pltpu.VMEM((B, D_HALF), jnp.float32),
            pltpu.VMEM((B, D_HALF), jnp.float32),
            pltpu.VMEM((B, D_HALF), jnp.float32),
            pltpu.VMEM_SHARED((N_NODES, D_HALF), jnp.float32),
            pltpu.SemaphoreType.DMA,
            pltpu.SemaphoreType.DMA,
            pltpu.SemaphoreType.DMA,
            pltpu.SemaphoreType.DMA,
        ],
    )(_sc_body)
    return kfn(msg2, e, idx3, seg3, zeros)


def _cat_body(a_ref, b_ref, out_ref):
    out_ref[...] = jnp.concatenate([a_ref[...], b_ref[...]], axis=1)


def _cat_call(partial):
    blk = 1000
    nblk = N_NODES // blk
    return pl.pallas_call(
        _cat_body,
        grid=(nblk,),
        in_specs=[
            pl.BlockSpec((blk, D_HALF), lambda i: (i, 0)),
            pl.BlockSpec((blk, D_HALF), lambda i: (i + N_NODES // blk, 0)),
        ],
        out_specs=pl.BlockSpec((blk, D_HID), lambda i: (i, 0)),
        out_shape=jax.ShapeDtypeStruct((N_NODES, D_HID), jnp.float32),
    )(partial, partial)


def kernel(node, edge, seg_i, idx_j, W_node, W_e1, b_e1, W_e2, b_e2):
    idx32 = idx_j.astype(jnp.int32)
    seg32 = seg_i.astype(jnp.int32)
    msg2 = _msg_call(node, W_node)
    e = _mlp_call(edge.T, W_e1, b_e1, W_e2, b_e2)
    zeros = jnp.zeros((N_NODES, D_HALF), jnp.float32)
    return _sc_call(msg2, e, idx32.reshape(NS, NSTEPS, B),
                    seg32.reshape(NS, NSTEPS, B), zeros)
